# baseline (speedup 1.0000x reference)
import functools

import jax
import jax.numpy as jnp
from jax import lax
from jax.experimental import pallas as pl
from jax.experimental.pallas import tpu as pltpu
from jax.experimental.pallas import tpu_sc as plsc

_N = 10000
_NP = 10240
_E = 320000
_K = 50
_NB = 200
_NBH = 40
_RPT = _NP // 16
_ZR = 32


def _mesh():
    return plsc.VectorSubcoreMesh(core_axis_name="c", subcore_axis_name="s")


def _make_prop(d):
    @functools.partial(
        pl.kernel,
        mesh=_mesh(),
        out_type=jax.ShapeDtypeStruct((2, _NP, d), jnp.float32),
        scratch_types=[
            pltpu.VMEM((_NBH, _K), jnp.int32),
            pltpu.VMEM((_NBH, _K), jnp.int32),
            pltpu.VMEM((4, _K, d), jnp.float32),
            pltpu.VMEM((_ZR, d), jnp.float32),
            pltpu.VMEM_SHARED((_NP, d), jnp.float32),
            pltpu.SemaphoreType.DMA,
            pltpu.SemaphoreType.DMA,
            pltpu.SemaphoreType.DMA,
            pltpu.SemaphoreType.DMA,
            pltpu.SemaphoreType.DMA,
            pltpu.SemaphoreType.DMA,
            pltpu.SemaphoreType.DMA,
            pltpu.SemaphoreType.DMA,
        ],
    )
    def prop(tab_hbm, src_hbm, dst_hbm, out_hbm, src_t, dst_t, rbuf, zbuf, acc,
             g0, g1, g2, g3, s0, s1, s2, s3):
        c = lax.axis_index("c")
        s = lax.axis_index("s")
        wid = c * 16 + s

        def zrow(i, carry):
            for j in range(d // 16):
                zbuf[i, pl.ds(j * 16, 16)] = jnp.zeros((16,), jnp.float32)
            return carry

        lax.fori_loop(0, _ZR, zrow, None)

        def zacc(i, carry):
            pltpu.sync_copy(zbuf, acc.at[pl.ds(s * _RPT + i * _ZR, _ZR), :])
            return carry

        lax.fori_loop(0, _RPT // _ZR, zacc, None)

        plsc.subcore_barrier()

        semg = (g0, g1, g2, g3)
        sems = (s0, s1, s2, s3)
        for phase in range(_NB // _NBH):
            base = wid * _NB + phase * _NBH
            pltpu.sync_copy(src_hbm.at[pl.ds(base, _NBH), :], src_t)
            pltpu.sync_copy(dst_hbm.at[pl.ds(base, _NBH), :], dst_t)

            pltpu.async_copy(tab_hbm.at[src_t.at[0]], rbuf.at[0], semg[0])
            pltpu.async_copy(tab_hbm.at[src_t.at[1]], rbuf.at[1], semg[1])

            def quad(i, carry):
                for j in range(4):
                    b = i * 4 + j
                    bg = (j + 2) % 4

                    @pl.when(b >= 2)
                    def _():
                        pltpu.make_async_copy(
                            rbuf.at[bg], acc.at[dst_t.at[b]], sems[bg]
                        ).wait()

                    @pl.when(b + 2 < _NBH)
                    def _():
                        pltpu.async_copy(
                            tab_hbm.at[src_t.at[b + 2]], rbuf.at[bg], semg[bg]
                        )

                    pltpu.make_async_copy(
                        tab_hbm.at[src_t.at[b]], rbuf.at[j], semg[j]
                    ).wait()
                    pltpu.async_copy(
                        rbuf.at[j], acc.at[dst_t.at[b]], sems[j], add=True
                    )
                return carry

            lax.fori_loop(0, _NBH // 4, quad, None)
            pltpu.make_async_copy(rbuf.at[2], acc.at[dst_t.at[0]], sems[2]).wait()
            pltpu.make_async_copy(rbuf.at[3], acc.at[dst_t.at[0]], sems[3]).wait()

        plsc.subcore_barrier()

        pltpu.sync_copy(
            acc.at[pl.ds(s * _RPT, _RPT), :],
            out_hbm.at[c, pl.ds(s * _RPT, _RPT), :],
        )

    return prop


@functools.partial(
    pl.kernel,
    mesh=_mesh(),
    out_type=jax.ShapeDtypeStruct((2, _NP, 128), jnp.float32),
    scratch_types=[
        pltpu.VMEM((_NB, _K), jnp.int32),
        pltpu.VMEM((_K, 128), jnp.float32),
        pltpu.VMEM((_ZR, 128), jnp.float32),
        pltpu.VMEM_SHARED((_NP, 128), jnp.float32),
    ],
)
def _deg_nogather(ones_hbm, zeros_hbm, dst_hbm, out_hbm, dst_t, ones_t, zbuf, acc):
    c = lax.axis_index("c")
    s = lax.axis_index("s")
    wid = c * 16 + s

    pltpu.sync_copy(ones_hbm, ones_t)
    pltpu.sync_copy(zeros_hbm, zbuf)

    def zacc(i, carry):
        pltpu.sync_copy(zbuf, acc.at[pl.ds(s * _RPT + i * _ZR, _ZR), :])
        return carry

    lax.fori_loop(0, _RPT // _ZR, zacc, None)

    pltpu.sync_copy(dst_hbm.at[pl.ds(wid * _NB, _NB), :], dst_t)
    plsc.subcore_barrier()

    def body(b, carry):
        pltpu.sync_copy(ones_t, acc.at[dst_t.at[b]], add=True)
        return carry

    lax.fori_loop(0, _NB, body, None)
    plsc.subcore_barrier()

    pltpu.sync_copy(
        acc.at[pl.ds(s * _RPT, _RPT), :],
        out_hbm.at[c, pl.ds(s * _RPT, _RPT), :],
    )


_prop128 = _make_prop(128)


def _tc_mm_body(x, w1, h_o):
    h_o[...] = jnp.dot(x[...], w1[...], preferred_element_type=jnp.float32)


def _tc_scale_body(degp, h, a1_o, dis_o):
    deg = degp[0, 0:_N, 0:1] + degp[1, 0:_N, 0:1] + 1.0
    dis = lax.rsqrt(deg)
    a1_o[...] = dis * h[...]
    dis_o[...] = dis


def _tc_mid_body(accp, a_prev, dis, b, gam, bet, wn, out):
    dis_v = dis[...]
    pre = dis_v * (accp[0, 0:_N, :] + accp[1, 0:_N, :] + a_prev[...]) + b[...]
    m = jnp.mean(pre, axis=0, keepdims=True)
    v = jnp.mean((pre - m) * (pre - m), axis=0, keepdims=True)
    y = gam[...] * (pre - m) * lax.rsqrt(v + 1e-5) + bet[...]
    y = jnp.maximum(y, 0.0)
    out[...] = dis_v * jnp.dot(y, wn[...], preferred_element_type=jnp.float32)


def _tc_final_body(accp, a3, dis, b3, out):
    out[...] = dis[...] * (accp[0, 0:_N, :] + accp[1, 0:_N, :] + a3[...]) + b3[...]


def kernel(x, edge_index, W1, b1, g1, be1, W2, b2, g2, be2, W3, b3):
    src = edge_index[0].astype(jnp.int32).reshape(_E // _K, _K)
    dst = edge_index[1].astype(jnp.int32).reshape(_E // _K, _K)

    ones_rows = jnp.ones((_K, 128), jnp.float32)
    zero_rows = jnp.zeros((_ZR, 128), jnp.float32)
    degp = _deg_nogather(ones_rows, zero_rows, dst)

    h1 = pl.pallas_call(
        _tc_mm_body,
        out_shape=jax.ShapeDtypeStruct((_N, 128), jnp.float32),
    )(x, W1)

    a1, dis = pl.pallas_call(
        _tc_scale_body,
        out_shape=[
            jax.ShapeDtypeStruct((_N, 128), jnp.float32),
            jax.ShapeDtypeStruct((_N, 1), jnp.float32),
        ],
    )(degp, h1)

    acc1 = _prop128(a1, src, dst)
    a2 = pl.pallas_call(
        _tc_mid_body,
        out_shape=jax.ShapeDtypeStruct((_N, 128), jnp.float32),
    )(acc1, a1, dis, b1.reshape(1, -1), g1.reshape(1, -1), be1.reshape(1, -1), W2)

    acc2 = _prop128(a2, src, dst)
    w3p = jnp.pad(W3, ((0, 0), (0, 128 - W3.shape[1])))
    b3p = jnp.pad(b3, (0, 128 - b3.shape[0])).reshape(1, -1)
    a3 = pl.pallas_call(
        _tc_mid_body,
        out_shape=jax.ShapeDtypeStruct((_N, 128), jnp.float32),
    )(acc2, a2, dis, b2.reshape(1, -1), g2.reshape(1, -1), be2.reshape(1, -1), w3p)

    acc3 = _prop128(a3, src, dst)
    out = pl.pallas_call(
        _tc_final_body,
        out_shape=jax.ShapeDtypeStruct((_N, 128), jnp.float32),
    )(acc3, a3, dis, b3p)

    return out[:, :40]


# speedup vs baseline: 1.0435x; 1.0435x over previous
import functools

import jax
import jax.numpy as jnp
from jax import lax
from jax.experimental import pallas as pl
from jax.experimental.pallas import tpu as pltpu
from jax.experimental.pallas import tpu_sc as plsc

_N = 10000
_NP = 10240
_E = 320000
_K = 125
_NB = 80
_NBH = 40
_RPT = _NP // 16
_ZR = 32


def _mesh():
    return plsc.VectorSubcoreMesh(core_axis_name="c", subcore_axis_name="s")


def _make_prop(d):
    @functools.partial(
        pl.kernel,
        mesh=_mesh(),
        out_type=jax.ShapeDtypeStruct((2, _NP, d), jnp.float32),
        scratch_types=[
            pltpu.VMEM((_NBH, _K), jnp.int32),
            pltpu.VMEM((_NBH, _K), jnp.int32),
            pltpu.VMEM((2, _K, d), jnp.float32),
            pltpu.VMEM((_ZR, d), jnp.float32),
            pltpu.VMEM_SHARED((_NP, d), jnp.float32),
            pltpu.SemaphoreType.DMA,
            pltpu.SemaphoreType.DMA,
        ],
    )
    def prop(tab_hbm, src_hbm, dst_hbm, out_hbm, src_t, dst_t, rbuf, zbuf, acc,
             sem0, sem1):
        c = lax.axis_index("c")
        s = lax.axis_index("s")
        wid = c * 16 + s

        def zrow(i, carry):
            for j in range(d // 16):
                zbuf[i, pl.ds(j * 16, 16)] = jnp.zeros((16,), jnp.float32)
            return carry

        lax.fori_loop(0, _ZR, zrow, None)

        def zacc(i, carry):
            pltpu.sync_copy(zbuf, acc.at[pl.ds(s * _RPT + i * _ZR, _ZR), :])
            return carry

        lax.fori_loop(0, _RPT // _ZR, zacc, None)

        plsc.subcore_barrier()

        for phase in range(_NB // _NBH):
            base = wid * _NB + phase * _NBH
            pltpu.sync_copy(src_hbm.at[pl.ds(base, _NBH), :], src_t)
            pltpu.sync_copy(dst_hbm.at[pl.ds(base, _NBH), :], dst_t)

            pltpu.async_copy(tab_hbm.at[src_t.at[0]], rbuf.at[0], sem0)
            pltpu.async_copy(tab_hbm.at[src_t.at[1]], rbuf.at[1], sem1)

            def stage(b, buf, sem):
                pltpu.make_async_copy(
                    tab_hbm.at[src_t.at[b]], rbuf.at[buf], sem
                ).wait()
                pltpu.sync_copy(rbuf.at[buf], acc.at[dst_t.at[b]], add=True)

                @pl.when(b + 2 < _NBH)
                def _():
                    pltpu.async_copy(
                        tab_hbm.at[src_t.at[b + 2]], rbuf.at[buf], sem
                    )

            def body(b, carry):
                even = lax.rem(b, 2) == 0

                @pl.when(even)
                def _():
                    stage(b, 0, sem0)

                @pl.when(jnp.logical_not(even))
                def _():
                    stage(b, 1, sem1)

                return carry

            lax.fori_loop(0, _NBH, body, None)

        plsc.subcore_barrier()

        pltpu.sync_copy(
            acc.at[pl.ds(s * _RPT, _RPT), :],
            out_hbm.at[c, pl.ds(s * _RPT, _RPT), :],
        )

    return prop


@functools.partial(
    pl.kernel,
    mesh=_mesh(),
    out_type=jax.ShapeDtypeStruct((2, _NP, 128), jnp.float32),
    scratch_types=[
        pltpu.VMEM((_NB, _K), jnp.int32),
        pltpu.VMEM((_K, 128), jnp.float32),
        pltpu.VMEM((_ZR, 128), jnp.float32),
        pltpu.VMEM_SHARED((_NP, 128), jnp.float32),
    ],
)
def _deg_nogather(ones_hbm, zeros_hbm, dst_hbm, out_hbm, dst_t, ones_t, zbuf, acc):
    c = lax.axis_index("c")
    s = lax.axis_index("s")
    wid = c * 16 + s

    pltpu.sync_copy(ones_hbm, ones_t)
    pltpu.sync_copy(zeros_hbm, zbuf)

    def zacc(i, carry):
        pltpu.sync_copy(zbuf, acc.at[pl.ds(s * _RPT + i * _ZR, _ZR), :])
        return carry

    lax.fori_loop(0, _RPT // _ZR, zacc, None)

    pltpu.sync_copy(dst_hbm.at[pl.ds(wid * _NB, _NB), :], dst_t)
    plsc.subcore_barrier()

    def body(b, carry):
        pltpu.sync_copy(ones_t, acc.at[dst_t.at[b]], add=True)
        return carry

    lax.fori_loop(0, _NB, body, None)
    plsc.subcore_barrier()

    pltpu.sync_copy(
        acc.at[pl.ds(s * _RPT, _RPT), :],
        out_hbm.at[c, pl.ds(s * _RPT, _RPT), :],
    )


_prop128 = _make_prop(128)


def _tc_first_body(degp, x, w1, a1_o, dis_o):
    deg = degp[0, 0:_N, 0:1] + degp[1, 0:_N, 0:1] + 1.0
    dis = lax.rsqrt(deg)
    h = jnp.dot(x[...], w1[...], preferred_element_type=jnp.float32)
    a1_o[...] = dis * h
    dis_o[...] = dis


def _tc_mid_body(accp, a_prev, dis, b, gam, bet, wn, out):
    dis_v = dis[...]
    pre = dis_v * (accp[0, 0:_N, :] + accp[1, 0:_N, :] + a_prev[...]) + b[...]
    m = jnp.mean(pre, axis=0, keepdims=True)
    v = jnp.mean((pre - m) * (pre - m), axis=0, keepdims=True)
    y = gam[...] * (pre - m) * lax.rsqrt(v + 1e-5) + bet[...]
    y = jnp.maximum(y, 0.0)
    out[...] = dis_v * jnp.dot(y, wn[...], preferred_element_type=jnp.float32)


def _tc_final_body(accp, a3, dis, b3, out):
    out[...] = dis[...] * (accp[0, 0:_N, :] + accp[1, 0:_N, :] + a3[...]) + b3[...]


def kernel(x, edge_index, W1, b1, g1, be1, W2, b2, g2, be2, W3, b3):
    src = edge_index[0].astype(jnp.int32).reshape(_E // _K, _K)
    dst = edge_index[1].astype(jnp.int32).reshape(_E // _K, _K)

    ones_rows = jnp.ones((_K, 128), jnp.float32)
    zero_rows = jnp.zeros((_ZR, 128), jnp.float32)
    degp = _deg_nogather(ones_rows, zero_rows, dst)

    a1, dis = pl.pallas_call(
        _tc_first_body,
        out_shape=[
            jax.ShapeDtypeStruct((_N, 128), jnp.float32),
            jax.ShapeDtypeStruct((_N, 1), jnp.float32),
        ],
    )(degp, x, W1)

    acc1 = _prop128(a1, src, dst)
    a2 = pl.pallas_call(
        _tc_mid_body,
        out_shape=jax.ShapeDtypeStruct((_N, 128), jnp.float32),
    )(acc1, a1, dis, b1.reshape(1, -1), g1.reshape(1, -1), be1.reshape(1, -1), W2)

    acc2 = _prop128(a2, src, dst)
    w3p = jnp.pad(W3, ((0, 0), (0, 128 - W3.shape[1])))
    b3p = jnp.pad(b3, (0, 128 - b3.shape[0])).reshape(1, -1)
    a3 = pl.pallas_call(
        _tc_mid_body,
        out_shape=jax.ShapeDtypeStruct((_N, 128), jnp.float32),
    )(acc2, a2, dis, b2.reshape(1, -1), g2.reshape(1, -1), be2.reshape(1, -1), w3p)

    acc3 = _prop128(a3, src, dst)
    out = pl.pallas_call(
        _tc_final_body,
        out_shape=jax.ShapeDtypeStruct((_N, 128), jnp.float32),
    )(acc3, a3, dis, b3p)

    return out[:, :40]


# speedup vs baseline: 1.0485x; 1.0048x over previous
import functools

import jax
import jax.numpy as jnp
from jax import lax
from jax.experimental import pallas as pl
from jax.experimental.pallas import tpu as pltpu
from jax.experimental.pallas import tpu_sc as plsc

_N = 10000
_NP = 10240
_E = 320000
_K = 125
_NB = 80
_NBH = 40
_RPT = _NP // 16
_ZR = 32


def _mesh():
    return plsc.VectorSubcoreMesh(core_axis_name="c", subcore_axis_name="s")


def _make_prop(d):
    @functools.partial(
        pl.kernel,
        mesh=_mesh(),
        out_type=jax.ShapeDtypeStruct((2, _NP, d), jnp.float32),
        scratch_types=[
            pltpu.VMEM((_NBH, _K), jnp.int32),
            pltpu.VMEM((_NBH, _K), jnp.int32),
            pltpu.VMEM((2, _K, d), jnp.float32),
            pltpu.VMEM((_ZR, d), jnp.float32),
            pltpu.VMEM_SHARED((_NP, d), jnp.float32),
            pltpu.SemaphoreType.DMA,
            pltpu.SemaphoreType.DMA,
        ],
    )
    def prop(tab_hbm, src_hbm, dst_hbm, out_hbm, src_t, dst_t, rbuf, zbuf, acc,
             sem0, sem1):
        c = lax.axis_index("c")
        s = lax.axis_index("s")
        wid = c * 16 + s

        def zrow(i, carry):
            for j in range(d // 16):
                zbuf[i, pl.ds(j * 16, 16)] = jnp.zeros((16,), jnp.float32)
            return carry

        lax.fori_loop(0, _ZR, zrow, None)

        def zacc(i, carry):
            pltpu.sync_copy(zbuf, acc.at[pl.ds(s * _RPT + i * _ZR, _ZR), :])
            return carry

        lax.fori_loop(0, _RPT // _ZR, zacc, None)

        plsc.subcore_barrier()

        for phase in range(_NB // _NBH):
            base = wid * _NB + phase * _NBH
            pltpu.sync_copy(src_hbm.at[pl.ds(base, _NBH), :], src_t)
            pltpu.sync_copy(dst_hbm.at[pl.ds(base, _NBH), :], dst_t)

            pltpu.async_copy(tab_hbm.at[src_t.at[0]], rbuf.at[0], sem0)
            pltpu.async_copy(tab_hbm.at[src_t.at[1]], rbuf.at[1], sem1)

            def stage(b, buf, sem):
                pltpu.make_async_copy(
                    tab_hbm.at[src_t.at[b]], rbuf.at[buf], sem
                ).wait()
                pltpu.sync_copy(rbuf.at[buf], acc.at[dst_t.at[b]], add=True)

                @pl.when(b + 2 < _NBH)
                def _():
                    pltpu.async_copy(
                        tab_hbm.at[src_t.at[b + 2]], rbuf.at[buf], sem
                    )

            def body(b, carry):
                even = lax.rem(b, 2) == 0

                @pl.when(even)
                def _():
                    stage(b, 0, sem0)

                @pl.when(jnp.logical_not(even))
                def _():
                    stage(b, 1, sem1)

                return carry

            lax.fori_loop(0, _NBH, body, None)

        plsc.subcore_barrier()

        pltpu.sync_copy(
            acc.at[pl.ds(s * _RPT, _RPT), :],
            out_hbm.at[c, pl.ds(s * _RPT, _RPT), :],
        )

    return prop


@functools.partial(
    pl.kernel,
    mesh=_mesh(),
    out_type=jax.ShapeDtypeStruct((2, _NP, 128), jnp.float32),
    scratch_types=[
        pltpu.VMEM((_NB, _K), jnp.int32),
        pltpu.VMEM((_K, 128), jnp.float32),
        pltpu.VMEM((_ZR, 128), jnp.float32),
        pltpu.VMEM_SHARED((_NP, 128), jnp.float32),
        pltpu.SemaphoreType.DMA,
    ],
)
def _deg_nogather(ones_hbm, zeros_hbm, dst_hbm, out_hbm, dst_t, ones_t, zbuf, acc,
                  semd):
    c = lax.axis_index("c")
    s = lax.axis_index("s")
    wid = c * 16 + s

    pltpu.sync_copy(ones_hbm, ones_t)
    pltpu.sync_copy(zeros_hbm, zbuf)

    def zacc(i, carry):
        pltpu.sync_copy(zbuf, acc.at[pl.ds(s * _RPT + i * _ZR, _ZR), :])
        return carry

    lax.fori_loop(0, _RPT // _ZR, zacc, None)

    pltpu.sync_copy(dst_hbm.at[pl.ds(wid * _NB, _NB), :], dst_t)
    plsc.subcore_barrier()

    def body(b, carry):
        pltpu.async_copy(ones_t, acc.at[dst_t.at[b]], semd, add=True)
        return carry

    lax.fori_loop(0, _NB, body, None)

    def drain(b, carry):
        pltpu.make_async_copy(ones_t, acc.at[dst_t.at[0]], semd).wait()
        return carry

    lax.fori_loop(0, _NB, drain, None)
    plsc.subcore_barrier()

    pltpu.sync_copy(
        acc.at[pl.ds(s * _RPT, _RPT), :],
        out_hbm.at[c, pl.ds(s * _RPT, _RPT), :],
    )


_prop128 = _make_prop(128)


def _tc_first_body(degp, x, w1, a1_o, dis_o):
    deg = degp[0, 0:_N, 0:1] + degp[1, 0:_N, 0:1] + 1.0
    dis = lax.rsqrt(deg)
    h = jnp.dot(x[...], w1[...], preferred_element_type=jnp.float32)
    a1_o[...] = dis * h
    dis_o[...] = dis


def _tc_mid_body(accp, a_prev, dis, b, gam, bet, wn, out):
    dis_v = dis[...]
    pre = dis_v * (accp[0, 0:_N, :] + accp[1, 0:_N, :] + a_prev[...]) + b[...]
    m = jnp.mean(pre, axis=0, keepdims=True)
    v = jnp.mean((pre - m) * (pre - m), axis=0, keepdims=True)
    y = gam[...] * (pre - m) * lax.rsqrt(v + 1e-5) + bet[...]
    y = jnp.maximum(y, 0.0)
    out[...] = dis_v * jnp.dot(y, wn[...], preferred_element_type=jnp.float32)


def _tc_final_body(accp, a3, dis, b3, out):
    out[...] = dis[...] * (accp[0, 0:_N, :] + accp[1, 0:_N, :] + a3[...]) + b3[...]


def kernel(x, edge_index, W1, b1, g1, be1, W2, b2, g2, be2, W3, b3):
    src = edge_index[0].astype(jnp.int32).reshape(_E // _K, _K)
    dst = edge_index[1].astype(jnp.int32).reshape(_E // _K, _K)

    ones_rows = jnp.ones((_K, 128), jnp.float32)
    zero_rows = jnp.zeros((_ZR, 128), jnp.float32)
    degp = _deg_nogather(ones_rows, zero_rows, dst)

    a1, dis = pl.pallas_call(
        _tc_first_body,
        out_shape=[
            jax.ShapeDtypeStruct((_N, 128), jnp.float32),
            jax.ShapeDtypeStruct((_N, 1), jnp.float32),
        ],
    )(degp, x, W1)

    acc1 = _prop128(a1, src, dst)
    a2 = pl.pallas_call(
        _tc_mid_body,
        out_shape=jax.ShapeDtypeStruct((_N, 128), jnp.float32),
    )(acc1, a1, dis, b1.reshape(1, -1), g1.reshape(1, -1), be1.reshape(1, -1), W2)

    acc2 = _prop128(a2, src, dst)
    w3p = jnp.pad(W3, ((0, 0), (0, 128 - W3.shape[1])))
    b3p = jnp.pad(b3, (0, 128 - b3.shape[0])).reshape(1, -1)
    a3 = pl.pallas_call(
        _tc_mid_body,
        out_shape=jax.ShapeDtypeStruct((_N, 128), jnp.float32),
    )(acc2, a2, dis, b2.reshape(1, -1), g2.reshape(1, -1), be2.reshape(1, -1), w3p)

    acc3 = _prop128(a3, src, dst)
    out = pl.pallas_call(
        _tc_final_body,
        out_shape=jax.ShapeDtypeStruct((_N, 128), jnp.float32),
    )(acc3, a3, dis, b3p)

    return out[:, :40]
